# baseline (speedup 1.0000x reference)
import functools

import jax
import jax.numpy as jnp
from jax import lax
from jax.experimental import pallas as pl
from jax.experimental.pallas import tpu as pltpu
from jax.experimental.pallas import tpu_sc as plsc

VOCAB_DIM = 64
SCALE = 8.0

_info = plsc.get_sparse_core_info()
NC, NS, L = _info.num_cores, _info.num_subcores, _info.num_lanes
NW = NC * NS

BATCH = 128
RING = 4


def _emb_body(nbpw, nbc, table_hbm, idx_hbm, out_hbm, idx_v, rows_in,
              rows_out, gsems, ssems):
    wid = lax.axis_index("s") * NC + lax.axis_index("c")
    base_b = wid * nbpw
    pltpu.sync_copy(idx_hbm.at[pl.ds(base_b, nbpw)], idx_v)
    iota = lax.iota(jnp.int32, L)

    def gather(b, r):
        return pltpu.make_async_copy(
            table_hbm.at[idx_v.at[b]], rows_in.at[r], gsems.at[r])

    def store(b, r):
        gb = base_b + b
        t = gb // nbc
        bc = gb - t * nbc
        return pltpu.make_async_copy(
            rows_out.at[r],
            out_hbm.at[pl.ds(t * 8, 8), bc],
            ssems.at[r])

    for r in range(RING):
        gather(r, r).start()

    def cycle(g, carry):
        for r in range(RING):
            b = g * RING + r
            gather(b, r).wait()

            @pl.when(g > 0)
            def _wait_prev_store():
                store(b - RING, r).wait()

            @plsc.parallel_loop(0, 8 * 8 * (BATCH // L), unroll=8)
            def _txp(i):
                f = i >> 3
                j = i & 7
                cols = iota + j * L
                feat = jnp.broadcast_to(f, (L,))
                v = plsc.load_gather(rows_in.at[r], [cols, feat])
                rows_out[r, f >> 3, f & 7, pl.ds(j * L, L)] = v * SCALE

            store(b, r).start()

            @pl.when(b + RING < nbpw)
            def _next_gather():
                gather(b + RING, r).start()
        return carry

    lax.fori_loop(0, nbpw // RING, cycle, 0)
    for r in range(RING):
        store(nbpw - RING + r, r).wait()


def kernel(x, table):
    b_dim, t_dim = x.shape
    n_rows = b_dim * t_dim
    nbc = b_dim // BATCH
    n_batches = n_rows // BATCH
    nbpw = n_batches // NW
    assert n_batches % (NW * RING) == 0 and VOCAB_DIM == 64

    idx = x.T.reshape(n_batches, BATCH).astype(jnp.int32)

    mesh = plsc.VectorSubcoreMesh(core_axis_name="c", subcore_axis_name="s")
    k = pl.kernel(
        functools.partial(_emb_body, nbpw, nbc),
        mesh=mesh,
        out_type=jax.ShapeDtypeStruct((t_dim * 8, nbc, 8, BATCH), jnp.float32),
        scratch_types=[
            pltpu.VMEM((nbpw, BATCH), jnp.int32),
            pltpu.VMEM((RING, BATCH, VOCAB_DIM), jnp.float32),
            pltpu.VMEM((RING, 8, 8, BATCH), jnp.float32),
            pltpu.SemaphoreType.DMA((RING,)),
            pltpu.SemaphoreType.DMA((RING,)),
        ],
        compiler_params=pltpu.CompilerParams(
            use_tc_tiling_on_sc=False, needs_layout_passes=False),
    )
    out_lin = k(table, idx)
    out = (out_lin.reshape(t_dim, 8, nbc, 8, BATCH)
           .transpose(2, 4, 0, 1, 3)
           .reshape(b_dim, t_dim, VOCAB_DIM))
    return out


# speedup vs baseline: 2.1912x; 2.1912x over previous
import functools

import jax
import jax.numpy as jnp
from jax import lax
from jax.experimental import pallas as pl
from jax.experimental.pallas import tpu as pltpu
from jax.experimental.pallas import tpu_sc as plsc

VOCAB_DIM = 64
SCALE = 8.0

_info = plsc.get_sparse_core_info()
NC, NS, L = _info.num_cores, _info.num_subcores, _info.num_lanes
NW = NC * NS

BATCH = 128
RING = 4
PADW = 129


def _emb_body(nbpw, nbc, table_hbm, idx_hbm, out_hbm, idx_v, rows_in,
              rows_out, gsems, ssems):
    wid = lax.axis_index("s") * NC + lax.axis_index("c")
    base_b = wid * nbpw
    pltpu.sync_copy(idx_hbm.at[pl.ds(base_b, nbpw)], idx_v)
    iota = lax.iota(jnp.int32, L)

    @plsc.parallel_loop(0, nbpw * (BATCH // L), unroll=8)
    def _dbl(i):
        bb = i >> 3
        k = (i & 7) * L
        v = idx_v[bb, pl.ds(k, L)]
        idx_v[bb, pl.ds(k, L)] = v + v

    def gather(b, r):
        return pltpu.make_async_copy(
            table_hbm.at[idx_v.at[b]], rows_in.at[r], gsems.at[r])

    def store(b, r):
        gb = base_b + b
        t = gb // nbc
        bc = gb - t * nbc
        return pltpu.make_async_copy(
            rows_out.at[r, :, :, pl.ds(0, BATCH)],
            out_hbm.at[pl.ds(t * 8, 8), bc],
            ssems.at[r])

    for r in range(RING):
        gather(r, r).start()

    dt_ds = []
    for j in range(VOCAB_DIM // L):
        d16 = iota + j * L
        dt_ds.append((d16 >> 3, d16 & 7))

    def cycle(g, carry):
        for r in range(RING):
            b = g * RING + r
            gather(b, r).wait()

            @pl.when(g > 0)
            def _wait_prev_store():
                store(b - RING, r).wait()

            for j in range(VOCAB_DIM // L):
                dtv, dsv = dt_ds[j]

                @plsc.parallel_loop(0, BATCH, unroll=8)
                def _txp(bl):
                    v = rows_in[r, bl, pl.ds(j * L, L)]
                    blv = jnp.broadcast_to(bl, (L,))
                    plsc.store_scatter(rows_out.at[r], [dtv, dsv, blv],
                                       v * SCALE)

            store(b, r).start()

            @pl.when(b + RING < nbpw)
            def _next_gather():
                gather(b + RING, r).start()
        return carry

    lax.fori_loop(0, nbpw // RING, cycle, 0)
    for r in range(RING):
        store(nbpw - RING + r, r).wait()


def kernel(x, table):
    b_dim, t_dim = x.shape
    n_rows = b_dim * t_dim
    nbc = b_dim // BATCH
    n_batches = n_rows // BATCH
    nbpw = n_batches // NW
    assert n_batches % (NW * RING) == 0 and VOCAB_DIM == 64

    tbl2 = jnp.pad(table, ((0, 0), (0, VOCAB_DIM))).reshape(-1, VOCAB_DIM)

    idx = x.T.reshape(n_batches, BATCH).astype(jnp.int32)

    mesh = plsc.VectorSubcoreMesh(core_axis_name="c", subcore_axis_name="s")
    k = pl.kernel(
        functools.partial(_emb_body, nbpw, nbc),
        mesh=mesh,
        out_type=jax.ShapeDtypeStruct((t_dim * 8, nbc, 8, BATCH), jnp.float32),
        scratch_types=[
            pltpu.VMEM((nbpw, BATCH), jnp.int32),
            pltpu.VMEM((RING, BATCH, VOCAB_DIM), jnp.float32),
            pltpu.VMEM((RING, 8, 8, PADW), jnp.float32),
            pltpu.SemaphoreType.DMA((RING,)),
            pltpu.SemaphoreType.DMA((RING,)),
        ],
        compiler_params=pltpu.CompilerParams(
            use_tc_tiling_on_sc=False, needs_layout_passes=False),
    )
    out_lin = k(tbl2, idx)
    out = (out_lin.reshape(t_dim, 8, nbc, 8, BATCH)
           .transpose(2, 4, 0, 1, 3)
           .reshape(b_dim, t_dim, VOCAB_DIM))
    return out
